# baseline (speedup 1.0000x reference)
import functools

import jax
import jax.numpy as jnp
from jax import lax
from jax.experimental import pallas as pl
from jax.experimental.pallas import tpu as pltpu
from jax.experimental.pallas import tpu_sc as plsc

_CH = 112


def _make_deg_kernel(n_bins, n_chunks_per_worker):
    info = plsc.get_sparse_core_info()
    nc, ns = info.num_cores, info.num_subcores
    nj = n_chunks_per_worker
    bpt = n_bins // ns
    mesh = plsc.VectorSubcoreMesh(core_axis_name="c", subcore_axis_name="s")

    def body(dst_hbm, out_hbm, idx_d, ones_v, zeros_v, deg_sh, sa, sb):
        cid = lax.axis_index("c")
        sid = lax.axis_index("s")
        w = sid * nc + cid
        pltpu.sync_copy(dst_hbm.at[pl.ds(w * nj, nj)], idx_d)
        for k in range(_CH // 16):
            ones_v[pl.ds(16 * k, 16)] = jnp.ones((16,), jnp.float32)
        for k in range(bpt // 16):
            zeros_v[pl.ds(16 * k, 16)] = jnp.zeros((16,), jnp.float32)
        pltpu.sync_copy(zeros_v, deg_sh.at[pl.ds(sid * bpt, bpt)])
        plsc.subcore_barrier()

        pltpu.async_copy(ones_v, deg_sh.at[idx_d.at[0]], sa, add=True)
        pltpu.async_copy(ones_v, deg_sh.at[idx_d.at[1]], sb, add=True)

        def step(i, carry):
            j = 2 * i
            pltpu.make_async_copy(ones_v, deg_sh.at[idx_d.at[j]], sa).wait()
            pltpu.async_copy(ones_v, deg_sh.at[idx_d.at[j + 2]], sa, add=True)
            pltpu.make_async_copy(ones_v, deg_sh.at[idx_d.at[j + 1]], sb).wait()
            pltpu.async_copy(ones_v, deg_sh.at[idx_d.at[j + 3]], sb, add=True)
            return carry

        lax.fori_loop(0, nj // 2 - 1, step, 0)
        pltpu.make_async_copy(ones_v, deg_sh.at[idx_d.at[nj - 2]], sa).wait()
        pltpu.make_async_copy(ones_v, deg_sh.at[idx_d.at[nj - 1]], sb).wait()
        plsc.subcore_barrier()
        pltpu.sync_copy(deg_sh.at[pl.ds(sid * bpt, bpt)],
                        out_hbm.at[pl.ds(cid * n_bins + sid * bpt, bpt)])

    return pl.kernel(
        body,
        out_type=jax.ShapeDtypeStruct((nc * n_bins,), jnp.float32),
        mesh=mesh,
        scratch_types=[
            pltpu.VMEM((nj, _CH), jnp.int32),
            pltpu.VMEM((_CH,), jnp.float32),
            pltpu.VMEM((bpt,), jnp.float32),
            pltpu.VMEM_SHARED((n_bins,), jnp.float32),
            pltpu.SemaphoreType.DMA,
            pltpu.SemaphoreType.DMA,
        ],
        compiler_params=pltpu.CompilerParams(use_tc_tiling_on_sc=False),
    )


_K = 3


def _make_agg_kernel(n, h, n_chunks_per_worker, n_acc_rows):
    info = plsc.get_sparse_core_info()
    nc, ns = info.num_cores, info.num_subcores
    nj = n_chunks_per_worker
    ng = nj // _K
    rpt = n // ns
    mesh = plsc.VectorSubcoreMesh(core_axis_name="c", subcore_axis_name="s")

    def body(z_hbm, src_hbm, dst_hbm, out0_hbm, out1_hbm,
             idx_s, idx_d, rows, acc_sh, gs0, gs1, gs2, ss0, ss1, ss2):
        cid = lax.axis_index("c")
        sid = lax.axis_index("s")
        w = sid * nc + cid
        pltpu.sync_copy(src_hbm.at[pl.ds(w * nj, nj)], idx_s)
        pltpu.sync_copy(dst_hbm.at[pl.ds(w * nj, nj)], idx_d)

        @pl.when(cid == 0)
        def _():
            pltpu.sync_copy(z_hbm.at[pl.ds(sid * rpt, rpt)],
                            acc_sh.at[pl.ds(sid * rpt, rpt)])

        @pl.when(cid == 1)
        def _():
            zb = rows.at[0, 0]

            def zfill(i, carry):
                for k in range(h // 16):
                    zb[i, pl.ds(16 * k, 16)] = jnp.zeros((16,), jnp.float32)
                return carry

            lax.fori_loop(0, _CH, zfill, 0)
            for m in range(rpt // _CH):
                pltpu.sync_copy(zb, acc_sh.at[pl.ds(sid * rpt + m * _CH, _CH)])
            rem = rpt % _CH
            if rem:
                pltpu.sync_copy(
                    zb.at[pl.ds(0, rem)],
                    acc_sh.at[pl.ds(sid * rpt + (rpt // _CH) * _CH, rem)])

        plsc.subcore_barrier()

        gsems = (gs0, gs1, gs2)
        ssems = (ss0, ss1, ss2)

        def gath(g, hb):
            for b in range(_K):
                pltpu.async_copy(z_hbm.at[idx_s.at[_K * g + b]],
                                 rows.at[hb, b], gsems[hb])

        def gath_wait(g, hb):
            for b in range(_K):
                pltpu.make_async_copy(z_hbm.at[idx_s.at[_K * g + b]],
                                      rows.at[hb, b], gsems[hb]).wait()

        def scat(g, hb):
            for b in range(_K):
                pltpu.async_copy(rows.at[hb, b], acc_sh.at[idx_d.at[_K * g + b]],
                                 ssems[hb], add=True)

        def scat_wait(g, hb):
            for b in range(_K):
                pltpu.make_async_copy(rows.at[hb, b],
                                      acc_sh.at[idx_d.at[_K * g + b]],
                                      ssems[hb]).wait()

        gath(0, 0)
        gath(1, 1)
        gath_wait(0, 0)
        scat(0, 0)
        gath(2, 2)
        gath_wait(1, 1)
        scat(1, 1)

        def step(i, carry):
            g0 = 3 * i + 2
            for u in range(3):
                v = g0 + u
                hf = u
                scat_wait(v - 2, hf)
                gath(v + 1, hf)
                gath_wait(v, (u + 2) % 3)
                scat(v, (u + 2) % 3)
            return carry

        lax.fori_loop(0, (ng - 3) // 3, step, 0)
        gl = ng - 1
        scat_wait(gl - 2, (gl + 1) % 3)
        gath_wait(gl, gl % 3)
        scat(gl, gl % 3)
        scat_wait(gl - 1, (gl - 1) % 3)
        scat_wait(gl, gl % 3)

        plsc.subcore_barrier()

        @pl.when(cid == 0)
        def _():
            pltpu.sync_copy(acc_sh.at[pl.ds(sid * rpt, rpt)],
                            out0_hbm.at[pl.ds(sid * rpt, rpt)])

        @pl.when(cid == 1)
        def _():
            pltpu.sync_copy(acc_sh.at[pl.ds(sid * rpt, rpt)],
                            out1_hbm.at[pl.ds(sid * rpt, rpt)])

    return pl.kernel(
        body,
        out_type=[jax.ShapeDtypeStruct((n, h), jnp.float32),
                  jax.ShapeDtypeStruct((n, h), jnp.float32)],
        mesh=mesh,
        scratch_types=[
            pltpu.VMEM((nj, _CH), jnp.int32),
            pltpu.VMEM((nj, _CH), jnp.int32),
            pltpu.VMEM((3, _K, _CH, h), jnp.float32),
            pltpu.VMEM_SHARED((n_acc_rows, h), jnp.float32),
            pltpu.SemaphoreType.DMA,
            pltpu.SemaphoreType.DMA,
            pltpu.SemaphoreType.DMA,
            pltpu.SemaphoreType.DMA,
            pltpu.SemaphoreType.DMA,
            pltpu.SemaphoreType.DMA,
        ],
        compiler_params=pltpu.CompilerParams(use_tc_tiling_on_sc=False),
    )


def _tc1_body(x_ref, w_ref, d0_ref, d1_ref, z_ref, dw_ref):
    hh = x_ref.shape[0], w_ref.shape[1] // 2
    deg = d0_ref[...] + d1_ref[...] + 1.0
    dinv2 = lax.rsqrt(deg)
    dw = jnp.concatenate(
        [jnp.broadcast_to(dinv2[:, 0:1], hh), jnp.broadcast_to(dinv2[:, 1:2], hh)],
        axis=1)
    dw_ref[...] = dw
    z_ref[...] = jnp.dot(x_ref[...], w_ref[...],
                         preferred_element_type=jnp.float32) * dw


def _tc2_body(p0_ref, p1_ref, dw_ref, b_ref, w_ref, z_ref):
    dw = dw_ref[...]
    hid = jnp.maximum(dw * (p0_ref[...] + p1_ref[...]) + b_ref[...], 0.0)
    z_ref[...] = jnp.dot(hid, w_ref[...],
                         preferred_element_type=jnp.float32) * dw


def _tc3_body(p0_ref, p1_ref, dw_ref, b_ref, z_ref):
    dw = dw_ref[...]
    hid = jnp.maximum(dw * (p0_ref[...] + p1_ref[...]) + b_ref[...], 0.0)
    z_ref[...] = hid * dw


def _tc4_body(p0_ref, p1_ref, dw_ref, w_ref, b_ref, o_ref):
    agg = dw_ref[...] * (p0_ref[...] + p1_ref[...])
    o_ref[...] = jnp.dot(agg, w_ref[...],
                         preferred_element_type=jnp.float32) + b_ref[...]


def _row_spec(bm, bk):
    return pl.BlockSpec((bm, bk), lambda i: (i, 0))


def _full_spec(shape):
    return pl.BlockSpec(shape, lambda i: (0,) * len(shape))


def kernel(x, edge_index, W1, b1, W2, b2, W3, b3):
    n, d = x.shape
    h = W1.shape[1]
    o = W3.shape[1]
    e = edge_index.shape[1]
    info = plsc.get_sparse_core_info()
    nc, ns = info.num_cores, info.num_subcores
    nw = nc * ns

    grain = _CH * nw * 3 * _K * 2
    ep = ((e + grain - 1) // grain) * grain
    npad = ep - e
    nj = ep // (_CH * nw)

    n_pad = ((n + 8 + 2047) // 2048) * 2048

    src = edge_index[0]
    dst = edge_index[1]
    if npad:
        pad_src = (jnp.arange(npad, dtype=jnp.int32) * 37) % n
        pad_dst = n + (jnp.arange(npad, dtype=jnp.int32) % (n_pad - n))
        src = jnp.concatenate([src, pad_src])
        dst = jnp.concatenate([dst, pad_dst])
    src2d = src.reshape(-1, _CH)
    dst2d = dst.reshape(-1, _CH)

    deg_k = _make_deg_kernel(n_pad, nj)
    agg_k = _make_agg_kernel(n_pad, h, nj, n_pad)

    deg2 = deg_k(dst2d)
    m = n_pad // 2
    d0p = deg2[:n_pad].reshape(m, 2)
    d1p = deg2[n_pad:].reshape(m, 2)

    xp = jnp.pad(x, ((0, n_pad - n), (0, 0))).reshape(m, 2 * d)
    zero_dh = jnp.zeros((d, h), jnp.float32)
    w1bd = jnp.block([[W1, zero_dh], [zero_dh, W1]])
    zero_hh = jnp.zeros((h, h), jnp.float32)
    w2bd = jnp.block([[W2, zero_hh], [zero_hh, W2]])
    zero_ho = jnp.zeros((h, o), jnp.float32)
    w3bd = jnp.block([[W3, zero_ho], [zero_ho, W3]])
    b1p = jnp.concatenate([b1, b1]).reshape(1, 2 * h)
    b2p = jnp.concatenate([b2, b2]).reshape(1, 2 * h)
    b3p = jnp.concatenate([b3, b3]).reshape(1, 2 * o)

    bmp = 512
    grid = (m // bmp,)

    z1p, dwp = pl.pallas_call(
        _tc1_body,
        grid=grid,
        in_specs=[_row_spec(bmp, 2 * d), _full_spec((2 * d, 2 * h)),
                  _row_spec(bmp, 2), _row_spec(bmp, 2)],
        out_specs=[_row_spec(bmp, 2 * h), _row_spec(bmp, 2 * h)],
        out_shape=[jax.ShapeDtypeStruct((m, 2 * h), jnp.float32),
                   jax.ShapeDtypeStruct((m, 2 * h), jnp.float32)],
    )(xp, w1bd, d0p, d1p)

    p0, p1 = agg_k(z1p.reshape(n_pad, h), src2d, dst2d)
    z2p = pl.pallas_call(
        _tc2_body,
        grid=grid,
        in_specs=[_row_spec(bmp, 2 * h), _row_spec(bmp, 2 * h),
                  _row_spec(bmp, 2 * h), _full_spec((1, 2 * h)),
                  _full_spec((2 * h, 2 * h))],
        out_specs=_row_spec(bmp, 2 * h),
        out_shape=jax.ShapeDtypeStruct((m, 2 * h), jnp.float32),
    )(p0.reshape(m, 2 * h), p1.reshape(m, 2 * h), dwp, b1p, w2bd)

    p0, p1 = agg_k(z2p.reshape(n_pad, h), src2d, dst2d)
    z3p = pl.pallas_call(
        _tc3_body,
        grid=grid,
        in_specs=[_row_spec(bmp, 2 * h), _row_spec(bmp, 2 * h),
                  _row_spec(bmp, 2 * h), _full_spec((1, 2 * h))],
        out_specs=_row_spec(bmp, 2 * h),
        out_shape=jax.ShapeDtypeStruct((m, 2 * h), jnp.float32),
    )(p0.reshape(m, 2 * h), p1.reshape(m, 2 * h), dwp, b2p)

    p0, p1 = agg_k(z3p.reshape(n_pad, h), src2d, dst2d)
    outp = pl.pallas_call(
        _tc4_body,
        grid=grid,
        in_specs=[_row_spec(bmp, 2 * h), _row_spec(bmp, 2 * h),
                  _row_spec(bmp, 2 * h), _full_spec((2 * h, 2 * o)),
                  _full_spec((1, 2 * o))],
        out_specs=_row_spec(bmp, 2 * o),
        out_shape=jax.ShapeDtypeStruct((m, 2 * o), jnp.float32),
    )(p0.reshape(m, 2 * h), p1.reshape(m, 2 * h), dwp, w3bd, b3p)

    return outp.reshape(n_pad, o)[:n]


# speedup vs baseline: 1.0314x; 1.0314x over previous
import jax
import jax.numpy as jnp
from jax import lax
from jax.experimental import pallas as pl
from jax.experimental.pallas import tpu as pltpu
from jax.experimental.pallas import tpu_sc as plsc

_CH = 112


def _make_deg_kernel(n_bins, n_chunks_per_worker):
    info = plsc.get_sparse_core_info()
    nc, ns = info.num_cores, info.num_subcores
    nj = n_chunks_per_worker
    bpt = n_bins // ns
    mesh = plsc.VectorSubcoreMesh(core_axis_name="c", subcore_axis_name="s")

    def body(dst_hbm, out_hbm, idx_d, ones_v, zeros_v, deg_sh, sa, sb):
        cid = lax.axis_index("c")
        sid = lax.axis_index("s")
        w = sid * nc + cid
        pltpu.sync_copy(dst_hbm.at[pl.ds(w * nj, nj)], idx_d)
        for k in range(_CH // 16):
            ones_v[pl.ds(16 * k, 16)] = jnp.ones((16,), jnp.float32)
        for k in range(bpt // 16):
            zeros_v[pl.ds(16 * k, 16)] = jnp.zeros((16,), jnp.float32)
        pltpu.sync_copy(zeros_v, deg_sh.at[pl.ds(sid * bpt, bpt)])
        plsc.subcore_barrier()

        pltpu.async_copy(ones_v, deg_sh.at[idx_d.at[0]], sa, add=True)
        pltpu.async_copy(ones_v, deg_sh.at[idx_d.at[1]], sb, add=True)

        def step(i, carry):
            j = 2 * i
            pltpu.make_async_copy(ones_v, deg_sh.at[idx_d.at[j]], sa).wait()
            pltpu.async_copy(ones_v, deg_sh.at[idx_d.at[j + 2]], sa, add=True)
            pltpu.make_async_copy(ones_v, deg_sh.at[idx_d.at[j + 1]], sb).wait()
            pltpu.async_copy(ones_v, deg_sh.at[idx_d.at[j + 3]], sb, add=True)
            return carry

        lax.fori_loop(0, nj // 2 - 1, step, 0)
        pltpu.make_async_copy(ones_v, deg_sh.at[idx_d.at[nj - 2]], sa).wait()
        pltpu.make_async_copy(ones_v, deg_sh.at[idx_d.at[nj - 1]], sb).wait()
        plsc.subcore_barrier()
        pltpu.sync_copy(deg_sh.at[pl.ds(sid * bpt, bpt)],
                        out_hbm.at[pl.ds(cid * n_bins + sid * bpt, bpt)])

    return pl.kernel(
        body,
        out_type=jax.ShapeDtypeStruct((nc * n_bins,), jnp.float32),
        mesh=mesh,
        scratch_types=[
            pltpu.VMEM((nj, _CH), jnp.int32),
            pltpu.VMEM((_CH,), jnp.float32),
            pltpu.VMEM((bpt,), jnp.float32),
            pltpu.VMEM_SHARED((n_bins,), jnp.float32),
            pltpu.SemaphoreType.DMA,
            pltpu.SemaphoreType.DMA,
        ],
        compiler_params=pltpu.CompilerParams(use_tc_tiling_on_sc=False),
    )


_K = 3


def _make_agg_kernel(n, h, n_chunks_per_worker, n_acc_rows):
    info = plsc.get_sparse_core_info()
    nc, ns = info.num_cores, info.num_subcores
    nj = n_chunks_per_worker
    ng = nj // _K
    rpt = n // ns
    mesh = plsc.VectorSubcoreMesh(core_axis_name="c", subcore_axis_name="s")

    def body(z_hbm, src_hbm, dst_hbm, out0_hbm, out1_hbm,
             idx_s, idx_d, rows, acc_sh, gs0, gs1, gs2, ss0, ss1, ss2):
        cid = lax.axis_index("c")
        sid = lax.axis_index("s")
        w = sid * nc + cid
        pltpu.sync_copy(src_hbm.at[pl.ds(w * nj, nj)], idx_s)
        pltpu.sync_copy(dst_hbm.at[pl.ds(w * nj, nj)], idx_d)

        @pl.when(cid == 0)
        def _():
            pltpu.sync_copy(z_hbm.at[pl.ds(sid * rpt, rpt)],
                            acc_sh.at[pl.ds(sid * rpt, rpt)])

        @pl.when(cid == 1)
        def _():
            zb = rows.at[0, 0]

            def zfill(i, carry):
                for k in range(h // 16):
                    zb[i, pl.ds(16 * k, 16)] = jnp.zeros((16,), jnp.float32)
                return carry

            lax.fori_loop(0, _CH, zfill, 0)
            for m in range(rpt // _CH):
                pltpu.sync_copy(zb, acc_sh.at[pl.ds(sid * rpt + m * _CH, _CH)])
            rem = rpt % _CH
            if rem:
                pltpu.sync_copy(
                    zb.at[pl.ds(0, rem)],
                    acc_sh.at[pl.ds(sid * rpt + (rpt // _CH) * _CH, rem)])

        plsc.subcore_barrier()

        gsems = (gs0, gs1, gs2)
        ssems = (ss0, ss1, ss2)

        def gath(g, hb):
            for b in range(_K):
                pltpu.async_copy(z_hbm.at[idx_s.at[_K * g + b]],
                                 rows.at[hb, b], gsems[hb])

        def gath_wait(g, hb):
            for b in range(_K):
                pltpu.make_async_copy(z_hbm.at[idx_s.at[_K * g + b]],
                                      rows.at[hb, b], gsems[hb]).wait()

        def scat(g, hb):
            for b in range(_K):
                pltpu.async_copy(rows.at[hb, b], acc_sh.at[idx_d.at[_K * g + b]],
                                 ssems[hb], add=True)

        def scat_wait(g, hb):
            for b in range(_K):
                pltpu.make_async_copy(rows.at[hb, b],
                                      acc_sh.at[idx_d.at[_K * g + b]],
                                      ssems[hb]).wait()

        gath(0, 0)
        gath(1, 1)
        gath_wait(0, 0)
        scat(0, 0)
        gath(2, 2)
        gath_wait(1, 1)
        scat(1, 1)

        def step(i, carry):
            g0 = 3 * i + 2
            for u in range(3):
                v = g0 + u
                hf = u
                scat_wait(v - 2, hf)
                gath(v + 1, hf)
                gath_wait(v, (u + 2) % 3)
                scat(v, (u + 2) % 3)
            return carry

        lax.fori_loop(0, (ng - 3) // 3, step, 0)
        gl = ng - 1
        scat_wait(gl - 2, (gl + 1) % 3)
        gath_wait(gl, gl % 3)
        scat(gl, gl % 3)
        scat_wait(gl - 1, (gl - 1) % 3)
        scat_wait(gl, gl % 3)

        plsc.subcore_barrier()

        @pl.when(cid == 0)
        def _():
            pltpu.sync_copy(acc_sh.at[pl.ds(sid * rpt, rpt)],
                            out0_hbm.at[pl.ds(sid * rpt, rpt)])

        @pl.when(cid == 1)
        def _():
            pltpu.sync_copy(acc_sh.at[pl.ds(sid * rpt, rpt)],
                            out1_hbm.at[pl.ds(sid * rpt, rpt)])

    return pl.kernel(
        body,
        out_type=[jax.ShapeDtypeStruct((n, h), jnp.float32),
                  jax.ShapeDtypeStruct((n, h), jnp.float32)],
        mesh=mesh,
        scratch_types=[
            pltpu.VMEM((nj, _CH), jnp.int32),
            pltpu.VMEM((nj, _CH), jnp.int32),
            pltpu.VMEM((3, _K, _CH, h), jnp.float32),
            pltpu.VMEM_SHARED((n_acc_rows, h), jnp.float32),
            pltpu.SemaphoreType.DMA,
            pltpu.SemaphoreType.DMA,
            pltpu.SemaphoreType.DMA,
            pltpu.SemaphoreType.DMA,
            pltpu.SemaphoreType.DMA,
            pltpu.SemaphoreType.DMA,
        ],
        compiler_params=pltpu.CompilerParams(use_tc_tiling_on_sc=False),
    )


def _tc1_body(x_ref, w_ref, d0_ref, d1_ref, z_ref, dw_ref):
    hh = x_ref.shape[0], w_ref.shape[1] // 2
    deg = d0_ref[...] + d1_ref[...] + 1.0
    dinv2 = lax.rsqrt(deg)
    dw = jnp.concatenate(
        [jnp.broadcast_to(dinv2[:, 0:1], hh), jnp.broadcast_to(dinv2[:, 1:2], hh)],
        axis=1)
    dw_ref[...] = dw
    z_ref[...] = jnp.dot(x_ref[...], w_ref[...],
                         preferred_element_type=jnp.float32) * dw


def _tc2_body(p0_ref, p1_ref, dw_ref, b_ref, w_ref, z_ref):
    dw = dw_ref[...]
    hid = jnp.maximum(dw * (p0_ref[...] + p1_ref[...]) + b_ref[...], 0.0)
    z_ref[...] = jnp.dot(hid, w_ref[...],
                         preferred_element_type=jnp.float32) * dw


def _tc3_body(p0_ref, p1_ref, dw_ref, b_ref, z_ref):
    dw = dw_ref[...]
    hid = jnp.maximum(dw * (p0_ref[...] + p1_ref[...]) + b_ref[...], 0.0)
    z_ref[...] = hid * dw


def _tc4_body(p0_ref, p1_ref, dw_ref, w_ref, b_ref, o_ref):
    agg = dw_ref[...] * (p0_ref[...] + p1_ref[...])
    o_ref[...] = jnp.dot(agg, w_ref[...],
                         preferred_element_type=jnp.float32) + b_ref[...]


def _row_spec(bm, bk):
    return pl.BlockSpec((bm, bk), lambda i: (i, 0))


def _full_spec(shape):
    return pl.BlockSpec(shape, lambda i: (0,) * len(shape))


def kernel(x, edge_index, W1, b1, W2, b2, W3, b3):
    n, d = x.shape
    h = W1.shape[1]
    o = W3.shape[1]
    e = edge_index.shape[1]
    info = plsc.get_sparse_core_info()
    nc, ns = info.num_cores, info.num_subcores
    nw = nc * ns

    grain = _CH * nw * 3 * _K * 2
    ep = ((e + grain - 1) // grain) * grain
    npad = ep - e
    nj = ep // (_CH * nw)

    n_pad = ((n + 8 + 2047) // 2048) * 2048

    src = edge_index[0]
    dst = edge_index[1]
    if npad:
        pad_src = (jnp.arange(npad, dtype=jnp.int32) * 37) % n
        pad_dst = n + (jnp.arange(npad, dtype=jnp.int32) % (n_pad - n))
        src = jnp.concatenate([src, pad_src])
        dst = jnp.concatenate([dst, pad_dst])
    src2d = src.reshape(-1, _CH)
    dst2d = dst.reshape(-1, _CH)

    deg_k = _make_deg_kernel(n_pad, nj)
    agg_k = _make_agg_kernel(n_pad, h, nj, n_pad)

    deg2 = deg_k(dst2d)
    m = n_pad // 2
    d0p = deg2[:n_pad].reshape(m, 2)
    d1p = deg2[n_pad:].reshape(m, 2)

    xp = jnp.pad(x, ((0, n_pad - n), (0, 0))).reshape(m, 2 * d)
    zero_dh = jnp.zeros((d, h), jnp.float32)
    w1bd = jnp.block([[W1, zero_dh], [zero_dh, W1]])
    zero_hh = jnp.zeros((h, h), jnp.float32)
    w2bd = jnp.block([[W2, zero_hh], [zero_hh, W2]])
    zero_ho = jnp.zeros((h, o), jnp.float32)
    w3bd = jnp.block([[W3, zero_ho], [zero_ho, W3]])
    b1p = jnp.concatenate([b1, b1]).reshape(1, 2 * h)
    b2p = jnp.concatenate([b2, b2]).reshape(1, 2 * h)
    b3p = jnp.concatenate([b3, b3]).reshape(1, 2 * o)

    bmp = 1024
    grid = (m // bmp,)

    z1p, dwp = pl.pallas_call(
        _tc1_body,
        grid=grid,
        in_specs=[_row_spec(bmp, 2 * d), _full_spec((2 * d, 2 * h)),
                  _row_spec(bmp, 2), _row_spec(bmp, 2)],
        out_specs=[_row_spec(bmp, 2 * h), _row_spec(bmp, 2 * h)],
        out_shape=[jax.ShapeDtypeStruct((m, 2 * h), jnp.float32),
                   jax.ShapeDtypeStruct((m, 2 * h), jnp.float32)],
    )(xp, w1bd, d0p, d1p)

    p0, p1 = agg_k(z1p.reshape(n_pad, h), src2d, dst2d)
    z2p = pl.pallas_call(
        _tc2_body,
        grid=grid,
        in_specs=[_row_spec(bmp, 2 * h), _row_spec(bmp, 2 * h),
                  _row_spec(bmp, 2 * h), _full_spec((1, 2 * h)),
                  _full_spec((2 * h, 2 * h))],
        out_specs=_row_spec(bmp, 2 * h),
        out_shape=jax.ShapeDtypeStruct((m, 2 * h), jnp.float32),
    )(p0.reshape(m, 2 * h), p1.reshape(m, 2 * h), dwp, b1p, w2bd)

    p0, p1 = agg_k(z2p.reshape(n_pad, h), src2d, dst2d)
    z3p = pl.pallas_call(
        _tc3_body,
        grid=grid,
        in_specs=[_row_spec(bmp, 2 * h), _row_spec(bmp, 2 * h),
                  _row_spec(bmp, 2 * h), _full_spec((1, 2 * h))],
        out_specs=_row_spec(bmp, 2 * h),
        out_shape=jax.ShapeDtypeStruct((m, 2 * h), jnp.float32),
    )(p0.reshape(m, 2 * h), p1.reshape(m, 2 * h), dwp, b2p)

    p0, p1 = agg_k(z3p.reshape(n_pad, h), src2d, dst2d)
    outp = pl.pallas_call(
        _tc4_body,
        grid=grid,
        in_specs=[_row_spec(bmp, 2 * h), _row_spec(bmp, 2 * h),
                  _row_spec(bmp, 2 * h), _full_spec((2 * h, 2 * o)),
                  _full_spec((1, 2 * o))],
        out_specs=_row_spec(bmp, 2 * o),
        out_shape=jax.ShapeDtypeStruct((m, 2 * o), jnp.float32),
    )(p0.reshape(m, 2 * h), p1.reshape(m, 2 * h), dwp, w3bd, b3p)

    return outp.reshape(n_pad, o)[:n]
